# Initial kernel scaffold; baseline (speedup 1.0000x reference)
import jax
import jax.numpy as jnp
from jax.experimental import pallas as pl


def kernel(x, W, b, row_indices, col_indices, values):
    raise NotImplementedError("write your pallas kernel here")


# speedup vs baseline: 1.3783x; 1.3783x over previous
import jax
import jax.numpy as jnp
from jax.experimental import pallas as pl
from jax.experimental.pallas import tpu as pltpu

_IN = 256
_OUT = 256
_NC = 6553

_CORES = 2
_BM = 4096


def _body(x_ref, w3_ref, b_ref, rows_ref, cols_ref, vals_ref, o_ref, weff_ref):
    j = pl.program_id(1)

    @pl.when(j == 0)
    def _build():
        weff_ref[...] = w3_ref[...]
        lane = jax.lax.broadcasted_iota(jnp.int32, (1, _IN), 1)

        def upd(i, carry):
            r = rows_ref[i]
            c = cols_ref[i]
            v = vals_ref[i]
            cur = weff_ref[r]
            base = w3_ref[r]
            weff_ref[r] = jnp.where(lane == c, base + 0.1 * v, cur)
            return carry

        jax.lax.fori_loop(0, _NC, upd, 0)

    weff = weff_ref[...].reshape(_OUT, _IN)
    acc = jax.lax.dot_general(
        x_ref[...], weff, (((1,), (1,)), ((), ())),
        preferred_element_type=jnp.float32)
    o_ref[...] = acc + b_ref[...]


def kernel(x, W, b, row_indices, col_indices, values):
    bsz, seq, d = x.shape
    rows_total = bsz * seq
    x2 = x.reshape(rows_total, d)
    w3 = W.reshape(_OUT, 1, _IN)
    b2 = b.reshape(1, _OUT)
    nstep = rows_total // (_CORES * _BM)

    out2 = pl.pallas_call(
        _body,
        out_shape=jax.ShapeDtypeStruct((rows_total, _OUT), x.dtype),
        grid=(_CORES, nstep),
        in_specs=[
            pl.BlockSpec((_BM, _IN), lambda c, j: (c * nstep + j, 0)),
            pl.BlockSpec((_OUT, 1, _IN), lambda c, j: (0, 0, 0)),
            pl.BlockSpec((1, _OUT), lambda c, j: (0, 0)),
            pl.BlockSpec(memory_space=pltpu.SMEM),
            pl.BlockSpec(memory_space=pltpu.SMEM),
            pl.BlockSpec(memory_space=pltpu.SMEM),
        ],
        out_specs=pl.BlockSpec((_BM, _OUT), lambda c, j: (c * nstep + j, 0)),
        scratch_shapes=[pltpu.VMEM((_OUT, 1, _IN), jnp.float32)],
        compiler_params=pltpu.CompilerParams(
            dimension_semantics=("parallel", "arbitrary"),
        ),
        name="small_world_layer",
    )(x2, w3, b2, row_indices, col_indices, values)
    return out2.reshape(bsz, seq, _OUT)


# speedup vs baseline: 1.8270x; 1.3256x over previous
import jax
import jax.numpy as jnp
from jax.experimental import pallas as pl
from jax.experimental.pallas import tpu as pltpu

_IN = 256
_OUT = 256
_NC = 6553

_BM = 8192


def _body(x_ref, w3_ref, b_ref, rows_ref, cols_ref, vals_ref, o_ref,
          weff3_ref, weff_ref):
    j = pl.program_id(0)

    @pl.when(j == 0)
    def _build():
        weff3_ref[...] = w3_ref[...]
        lane = jax.lax.broadcasted_iota(jnp.int32, (1, _IN), 1)

        def upd(i, carry):
            r = rows_ref[i]
            c = cols_ref[i]
            v = vals_ref[i]
            cur = weff3_ref[r]
            base = w3_ref[r]
            weff3_ref[r] = jnp.where(lane == c, base + 0.1 * v, cur)
            return carry

        jax.lax.fori_loop(0, _NC, upd, 0)
        weff_ref[...] = weff3_ref[...].reshape(_OUT, _IN)

    acc = jax.lax.dot_general(
        x_ref[...], weff_ref[...], (((1,), (1,)), ((), ())),
        preferred_element_type=jnp.float32)
    o_ref[...] = acc + b_ref[...]


def kernel(x, W, b, row_indices, col_indices, values):
    bsz, seq, d = x.shape
    rows_total = bsz * seq
    x2 = x.reshape(rows_total, d)
    w3 = W.reshape(_OUT, 1, _IN)
    b2 = b.reshape(1, _OUT)
    nstep = rows_total // _BM

    out2 = pl.pallas_call(
        _body,
        out_shape=jax.ShapeDtypeStruct((rows_total, _OUT), x.dtype),
        grid=(nstep,),
        in_specs=[
            pl.BlockSpec((_BM, _IN), lambda j: (j, 0)),
            pl.BlockSpec((_OUT, 1, _IN), lambda j: (0, 0, 0)),
            pl.BlockSpec((1, _OUT), lambda j: (0, 0)),
            pl.BlockSpec(memory_space=pltpu.SMEM),
            pl.BlockSpec(memory_space=pltpu.SMEM),
            pl.BlockSpec(memory_space=pltpu.SMEM),
        ],
        out_specs=pl.BlockSpec((_BM, _OUT), lambda j: (j, 0)),
        scratch_shapes=[pltpu.VMEM((_OUT, 1, _IN), jnp.float32),
                        pltpu.VMEM((_OUT, _IN), jnp.float32)],
        compiler_params=pltpu.CompilerParams(
            dimension_semantics=("arbitrary",),
            vmem_limit_bytes=56 * 1024 * 1024,
        ),
        name="small_world_layer",
    )(x2, w3, b2, row_indices, col_indices, values)
    return out2.reshape(bsz, seq, _OUT)


# speedup vs baseline: 2.2587x; 1.2363x over previous
import jax
import jax.numpy as jnp
from jax.experimental import pallas as pl
from jax.experimental.pallas import tpu as pltpu

_IN = 256
_OUT = 256
_NC = 6553
_CH = 128
_NCH = -(-_NC // _CH)

_BM = 8192


def _body(x_ref, w_ref, b_ref, fk_ref, vals_ref, o_ref, weff_ref):
    j = pl.program_id(0)

    @pl.when(j == 0)
    def _build():
        iota_o = jax.lax.broadcasted_iota(jnp.int32, (_OUT, _CH), 0)
        a_idx = jax.lax.broadcasted_iota(jnp.int32, (_CH, _CH), 0)
        b_idx = jax.lax.broadcasted_iota(jnp.int32, (_CH, _CH), 1)
        tri = jnp.where(a_idx > b_idx, 1.0, 0.0)

        weff_ref[...] = jnp.zeros((_OUT, _IN), jnp.float32)

        def chunk(t, carry):
            fk = fk_ref[t]
            vals = vals_ref[t]
            r = jax.lax.shift_right_logical(fk, 8)
            c = jnp.bitwise_and(fk, 255)
            ohr = jnp.where(iota_o == r, 1.0, 0.0)
            eqc = iota_o == c
            ohc = jnp.where(eqc, 1.0, 0.0)
            gr = jax.lax.dot_general(ohr, ohr, (((0,), (0,)), ((), ())),
                                     preferred_element_type=jnp.float32)
            gc = jax.lax.dot_general(ohc, ohc, (((0,), (0,)), ((), ())),
                                     preferred_element_type=jnp.float32)
            later_dup = jnp.sum(gr * gc * tri, axis=0, keepdims=True)
            keep = jnp.where(later_dup == 0.0, 1.0, 0.0)
            ohcv = jnp.where(eqc, 0.1 * vals * keep, 0.0)
            delta = jax.lax.dot_general(ohr, ohcv, (((1,), (1,)), ((), ())),
                                        preferred_element_type=jnp.float32)
            touched = jax.lax.dot_general(ohr, ohc, (((1,), (1,)), ((), ())),
                                          preferred_element_type=jnp.float32)
            weff_ref[...] = jnp.where(touched > 0.5, delta, weff_ref[...])
            return carry

        jax.lax.fori_loop(0, _NCH, chunk, 0)
        weff_ref[...] = weff_ref[...] + w_ref[...]

    acc = jax.lax.dot_general(
        x_ref[...], weff_ref[...], (((1,), (1,)), ((), ())),
        preferred_element_type=jnp.float32)
    o_ref[...] = acc + b_ref[...]


def kernel(x, W, b, row_indices, col_indices, values):
    bsz, seq, d = x.shape
    rows_total = bsz * seq
    x2 = x.reshape(rows_total, d)
    b2 = b.reshape(1, _OUT)

    fk = row_indices * _IN + col_indices
    pad = _NCH * _CH - _NC
    fk = jnp.concatenate([fk, jnp.broadcast_to(fk[-1:], (pad,))])
    vp = jnp.concatenate([values, jnp.broadcast_to(values[-1:], (pad,))])
    fk3 = fk.reshape(_NCH, 1, _CH)
    vals3 = vp.reshape(_NCH, 1, _CH).astype(jnp.float32)

    nstep = rows_total // _BM

    out2 = pl.pallas_call(
        _body,
        out_shape=jax.ShapeDtypeStruct((rows_total, _OUT), x.dtype),
        grid=(nstep,),
        in_specs=[
            pl.BlockSpec((_BM, _IN), lambda j: (j, 0)),
            pl.BlockSpec((_OUT, _IN), lambda j: (0, 0)),
            pl.BlockSpec((1, _OUT), lambda j: (0, 0)),
            pl.BlockSpec((_NCH, 1, _CH), lambda j: (0, 0, 0)),
            pl.BlockSpec((_NCH, 1, _CH), lambda j: (0, 0, 0)),
        ],
        out_specs=pl.BlockSpec((_BM, _OUT), lambda j: (j, 0)),
        scratch_shapes=[pltpu.VMEM((_OUT, _IN), jnp.float32)],
        compiler_params=pltpu.CompilerParams(
            dimension_semantics=("arbitrary",),
            vmem_limit_bytes=56 * 1024 * 1024,
        ),
        name="small_world_layer",
    )(x2, W, b2, fk3, vals3)
    return out2.reshape(bsz, seq, _OUT)


# speedup vs baseline: 2.3215x; 1.0278x over previous
import jax
import jax.numpy as jnp
from jax.experimental import pallas as pl
from jax.experimental.pallas import tpu as pltpu

_IN = 256
_OUT = 256
_NC = 6553
_CH = 128
_NCH = -(-_NC // _CH)

_BM = 8192


def _body(x_ref, w_ref, b_ref, fk_ref, vals_ref, o_ref, weff_ref):
    j = pl.program_id(0)

    @pl.when(j == 0)
    def _build():
        iota_o = jax.lax.broadcasted_iota(jnp.int32, (_OUT, _CH), 0)
        a_idx = jax.lax.broadcasted_iota(jnp.int32, (_CH, _CH), 0)
        b_idx = jax.lax.broadcasted_iota(jnp.int32, (_CH, _CH), 1)
        tri = jnp.where(a_idx > b_idx, 1.0, 0.0)

        weff_ref[...] = jnp.zeros((_OUT, _IN), jnp.float32)

        def chunk_delta(t):
            fk = fk_ref[t]
            vals = vals_ref[t]
            r = jax.lax.shift_right_logical(fk, 8)
            c = jnp.bitwise_and(fk, 255)
            ohr = jnp.where(iota_o == r, 1.0, 0.0)
            eqc = iota_o == c
            ohc = jnp.where(eqc, 1.0, 0.0)
            gr = jax.lax.dot_general(ohr, ohr, (((0,), (0,)), ((), ())),
                                     preferred_element_type=jnp.float32)
            gc = jax.lax.dot_general(ohc, ohc, (((0,), (0,)), ((), ())),
                                     preferred_element_type=jnp.float32)
            later_dup = jnp.sum(gr * gc * tri, axis=0, keepdims=True)
            keep = jnp.where(later_dup == 0.0, 1.0, 0.0)
            ohcv = jnp.where(eqc, 0.1 * vals * keep, 0.0)
            rhs = jnp.concatenate([ohcv, ohc], axis=0)
            both = jax.lax.dot_general(ohr, rhs, (((1,), (1,)), ((), ())),
                                       preferred_element_type=jnp.float32)
            return both[:, :_IN], both[:, _IN:]

        def chunk2(i, carry):
            d0, t0 = chunk_delta(2 * i)
            d1, t1 = chunk_delta(2 * i + 1)
            cur = weff_ref[...]
            weff_ref[...] = jnp.where(t1 > 0.5, d1,
                                      jnp.where(t0 > 0.5, d0, cur))
            return carry

        jax.lax.fori_loop(0, _NCH // 2, chunk2, 0)
        weff_ref[...] = weff_ref[...] + w_ref[...]

    acc = jax.lax.dot_general(
        x_ref[...], weff_ref[...], (((1,), (1,)), ((), ())),
        preferred_element_type=jnp.float32)
    o_ref[...] = acc + b_ref[...]


def kernel(x, W, b, row_indices, col_indices, values):
    bsz, seq, d = x.shape
    rows_total = bsz * seq
    x2 = x.reshape(rows_total, d)
    b2 = b.reshape(1, _OUT)

    fk = row_indices * _IN + col_indices
    pad = _NCH * _CH - _NC
    fk = jnp.concatenate([fk, jnp.broadcast_to(fk[-1:], (pad,))])
    vp = jnp.concatenate([values, jnp.broadcast_to(values[-1:], (pad,))])
    fk3 = fk.reshape(_NCH, 1, _CH)
    vals3 = vp.reshape(_NCH, 1, _CH).astype(jnp.float32)

    nstep = rows_total // _BM

    out2 = pl.pallas_call(
        _body,
        out_shape=jax.ShapeDtypeStruct((rows_total, _OUT), x.dtype),
        grid=(nstep,),
        in_specs=[
            pl.BlockSpec((_BM, _IN), lambda j: (j, 0)),
            pl.BlockSpec((_OUT, _IN), lambda j: (0, 0)),
            pl.BlockSpec((1, _OUT), lambda j: (0, 0)),
            pl.BlockSpec((_NCH, 1, _CH), lambda j: (0, 0, 0)),
            pl.BlockSpec((_NCH, 1, _CH), lambda j: (0, 0, 0)),
        ],
        out_specs=pl.BlockSpec((_BM, _OUT), lambda j: (j, 0)),
        scratch_shapes=[pltpu.VMEM((_OUT, _IN), jnp.float32)],
        compiler_params=pltpu.CompilerParams(
            dimension_semantics=("arbitrary",),
            vmem_limit_bytes=56 * 1024 * 1024,
        ),
        name="small_world_layer",
    )(x2, W, b2, fk3, vals3)
    return out2.reshape(bsz, seq, _OUT)


# speedup vs baseline: 2.3848x; 1.0273x over previous
import jax
import jax.numpy as jnp
from jax.experimental import pallas as pl
from jax.experimental.pallas import tpu as pltpu

_IN = 256
_OUT = 256
_NC = 6553
_CH = 128
_NCH = -(-_NC // _CH)

_BM = 8192


def _body(x_ref, w_ref, b_ref, fk_ref, fkt_ref, vals_ref, o_ref, weff_ref):
    j = pl.program_id(0)

    @pl.when(j == 0)
    def _build():
        iota_o = jax.lax.broadcasted_iota(jnp.int32, (_OUT, _CH), 0)
        a_idx = jax.lax.broadcasted_iota(jnp.int32, (_CH, _CH), 0)
        b_idx = jax.lax.broadcasted_iota(jnp.int32, (_CH, _CH), 1)
        tri = jnp.where(a_idx > b_idx, 1.0, 0.0)

        weff_ref[...] = jnp.zeros((_OUT, _IN), jnp.float32)

        def chunk_delta(t):
            fk = fk_ref[t]
            fkt = fkt_ref[t]
            vals = vals_ref[t]
            r = jax.lax.shift_right_logical(fk, 8)
            c = jnp.bitwise_and(fk, 255)
            ohr = jnp.where(iota_o == r, 1.0, 0.0)
            eqc = iota_o == c
            ohc = jnp.where(eqc, 1.0, 0.0)
            same = fkt == fk
            later_dup = jnp.sum(jnp.where(same, tri, 0.0), axis=0,
                                keepdims=True)
            keep = jnp.where(later_dup == 0.0, 1.0, 0.0)
            ohcv = jnp.where(eqc, 0.1 * vals * keep, 0.0)
            rhs = jnp.concatenate([ohcv, ohc], axis=0)
            both = jax.lax.dot_general(ohr, rhs, (((1,), (1,)), ((), ())),
                                       preferred_element_type=jnp.float32)
            return both[:, :_IN], both[:, _IN:]

        def chunk2(i, carry):
            d0, t0 = chunk_delta(2 * i)
            d1, t1 = chunk_delta(2 * i + 1)
            cur = weff_ref[...]
            weff_ref[...] = jnp.where(t1 > 0.5, d1,
                                      jnp.where(t0 > 0.5, d0, cur))
            return carry

        jax.lax.fori_loop(0, _NCH // 2, chunk2, 0)
        weff_ref[...] = weff_ref[...] + w_ref[...]

    acc = jax.lax.dot_general(
        x_ref[...], weff_ref[...], (((1,), (1,)), ((), ())),
        preferred_element_type=jnp.float32)
    o_ref[...] = acc + b_ref[...]


def kernel(x, W, b, row_indices, col_indices, values):
    bsz, seq, d = x.shape
    rows_total = bsz * seq
    x2 = x.reshape(rows_total, d)
    b2 = b.reshape(1, _OUT)

    fk = row_indices * _IN + col_indices
    pad = _NCH * _CH - _NC
    fk = jnp.concatenate([fk, jnp.broadcast_to(fk[-1:], (pad,))])
    vp = jnp.concatenate([values, jnp.broadcast_to(values[-1:], (pad,))])
    fk3 = fk.reshape(_NCH, 1, _CH)
    fkt3 = fk.reshape(_NCH, _CH, 1)
    vals3 = vp.reshape(_NCH, 1, _CH).astype(jnp.float32)

    nstep = rows_total // _BM

    out2 = pl.pallas_call(
        _body,
        out_shape=jax.ShapeDtypeStruct((rows_total, _OUT), x.dtype),
        grid=(nstep,),
        in_specs=[
            pl.BlockSpec((_BM, _IN), lambda j: (j, 0)),
            pl.BlockSpec((_OUT, _IN), lambda j: (0, 0)),
            pl.BlockSpec((1, _OUT), lambda j: (0, 0)),
            pl.BlockSpec((_NCH, 1, _CH), lambda j: (0, 0, 0)),
            pl.BlockSpec((_NCH, _CH, 1), lambda j: (0, 0, 0)),
            pl.BlockSpec((_NCH, 1, _CH), lambda j: (0, 0, 0)),
        ],
        out_specs=pl.BlockSpec((_BM, _OUT), lambda j: (j, 0)),
        scratch_shapes=[pltpu.VMEM((_OUT, _IN), jnp.float32)],
        compiler_params=pltpu.CompilerParams(
            dimension_semantics=("arbitrary",),
            vmem_limit_bytes=56 * 1024 * 1024,
        ),
        name="small_world_layer",
    )(x2, W, b2, fk3, fkt3, vals3)
    return out2.reshape(bsz, seq, _OUT)


# speedup vs baseline: 2.4907x; 1.0444x over previous
import jax
import jax.numpy as jnp
from jax.experimental import pallas as pl
from jax.experimental.pallas import tpu as pltpu

_IN = 256
_OUT = 256
_NC = 6553
_CH = 128
_NCH = -(-_NC // _CH)

_BM = 8192
_NSTEP = 16 * 16384 // _BM
_K = 4


def _body(x_ref, w_ref, b_ref, fk_ref, fkt_ref, vals_ref, o_ref, weff_ref,
          ring_ref, in_sems):
    j = pl.program_id(0)

    def start_load(step):
        slot = jax.lax.rem(step, _K)
        pltpu.make_async_copy(
            x_ref.at[pl.ds(pl.multiple_of(step * _BM, _BM), _BM), :],
            ring_ref.at[slot], in_sems.at[slot]).start()

    @pl.when(j == 0)
    def _prefetch():
        for s in range(_K - 1):
            start_load(s)

    @pl.when(j == 0)
    def _build():
        iota_o = jax.lax.broadcasted_iota(jnp.int32, (_OUT, _CH), 0)
        a_idx = jax.lax.broadcasted_iota(jnp.int32, (_CH, _CH), 0)
        b_idx = jax.lax.broadcasted_iota(jnp.int32, (_CH, _CH), 1)
        tri = jnp.where(a_idx > b_idx, 1.0, 0.0)

        weff_ref[...] = jnp.zeros((_OUT, _IN), jnp.float32)

        def chunk_delta(t):
            fk = fk_ref[t]
            fkt = fkt_ref[t]
            vals = vals_ref[t]
            r = jax.lax.shift_right_logical(fk, 8)
            c = jnp.bitwise_and(fk, 255)
            ohr = jnp.where(iota_o == r, 1.0, 0.0)
            eqc = iota_o == c
            ohc = jnp.where(eqc, 1.0, 0.0)
            same = fkt == fk
            later_dup = jnp.sum(jnp.where(same, tri, 0.0), axis=0,
                                keepdims=True)
            keep = jnp.where(later_dup == 0.0, 1.0, 0.0)
            ohcv = jnp.where(eqc, 0.1 * vals * keep, 0.0)
            rhs = jnp.concatenate([ohcv, ohc], axis=0)
            both = jax.lax.dot_general(ohr, rhs, (((1,), (1,)), ((), ())),
                                       preferred_element_type=jnp.float32)
            return both[:, :_IN], both[:, _IN:]

        def chunk2(i, carry):
            d0, t0 = chunk_delta(2 * i)
            d1, t1 = chunk_delta(2 * i + 1)
            cur = weff_ref[...]
            weff_ref[...] = jnp.where(t1 > 0.5, d1,
                                      jnp.where(t0 > 0.5, d0, cur))
            return carry

        jax.lax.fori_loop(0, _NCH // 2, chunk2, 0)
        weff_ref[...] = weff_ref[...] + w_ref[...]

    @pl.when(j + (_K - 1) < _NSTEP)
    def _refill():
        start_load(j + _K - 1)

    slot = jax.lax.rem(j, _K)
    pltpu.make_async_copy(
        x_ref.at[pl.ds(0, _BM), :], ring_ref.at[slot],
        in_sems.at[slot]).wait()
    acc = jax.lax.dot_general(
        ring_ref[slot], weff_ref[...], (((1,), (1,)), ((), ())),
        preferred_element_type=jnp.float32)
    o_ref[...] = acc + b_ref[...]


def kernel(x, W, b, row_indices, col_indices, values):
    bsz, seq, d = x.shape
    rows_total = bsz * seq
    x2 = x.reshape(rows_total, d)
    b2 = b.reshape(1, _OUT)

    fk = row_indices * _IN + col_indices
    pad = _NCH * _CH - _NC
    fk = jnp.concatenate([fk, jnp.broadcast_to(fk[-1:], (pad,))])
    vp = jnp.concatenate([values, jnp.broadcast_to(values[-1:], (pad,))])
    fk3 = fk.reshape(_NCH, 1, _CH)
    fkt3 = fk.reshape(_NCH, _CH, 1)
    vals3 = vp.reshape(_NCH, 1, _CH).astype(jnp.float32)

    nstep = rows_total // _BM

    out2 = pl.pallas_call(
        _body,
        out_shape=jax.ShapeDtypeStruct((rows_total, _OUT), x.dtype),
        grid=(nstep,),
        in_specs=[
            pl.BlockSpec(memory_space=pl.ANY),
            pl.BlockSpec((_OUT, _IN), lambda j: (0, 0)),
            pl.BlockSpec((1, _OUT), lambda j: (0, 0)),
            pl.BlockSpec((_NCH, 1, _CH), lambda j: (0, 0, 0)),
            pl.BlockSpec((_NCH, _CH, 1), lambda j: (0, 0, 0)),
            pl.BlockSpec((_NCH, 1, _CH), lambda j: (0, 0, 0)),
        ],
        out_specs=pl.BlockSpec((_BM, _OUT), lambda j: (j, 0)),
        scratch_shapes=[pltpu.VMEM((_OUT, _IN), jnp.float32),
                        pltpu.VMEM((_K, _BM, _IN), jnp.float32),
                        pltpu.SemaphoreType.DMA((_K,))],
        compiler_params=pltpu.CompilerParams(
            dimension_semantics=("arbitrary",),
            vmem_limit_bytes=56 * 1024 * 1024,
        ),
        name="small_world_layer",
    )(x2, W, b2, fk3, fkt3, vals3)
    return out2.reshape(bsz, seq, _OUT)
